# baseline (speedup 1.0000x reference)
import functools
import math

import jax
import jax.numpy as jnp
from jax import lax
from jax.experimental import pallas as pl
from jax.experimental.pallas import tpu as pltpu
from jax.experimental.pallas import tpu_sc as plsc

D_MODEL = 64
SCALE = math.sqrt(D_MODEL)
SEQ = 50

NUM_CORES = 2
NUM_SUBCORES = 16
NUM_WORKERS = NUM_CORES * NUM_SUBCORES
LANES = 16

BATCHES = 16384
BATCH_PER_WORKER = BATCHES // NUM_WORKERS
NB = 16
STEPS = BATCH_PER_WORKER // NB
CHUNK = NB * SEQ


def _emb_kernel(idx_hbm, lut_hbm, out_hbm, idx_v, rows0, rows1, gsem0, gsem1,
                osem0, osem1):
    wid = lax.axis_index("s") * NUM_CORES + lax.axis_index("c")
    b0 = wid * BATCH_PER_WORKER

    pltpu.sync_copy(idx_hbm.at[pl.ds(b0, BATCH_PER_WORKER)], idx_v)

    def gathers(buf, sem, s):
        return [
            pltpu.make_async_copy(lut_hbm.at[idx_v.at[s * NB + t]],
                                  buf.at[pl.ds(t * SEQ, SEQ)], sem)
            for t in range(NB)
        ]

    def scatters(buf, sem, s):
        return [
            pltpu.make_async_copy(buf.at[pl.ds(t * SEQ, SEQ)],
                                  out_hbm.at[b0 + s * NB + t], sem)
            for t in range(NB)
        ]

    def scale(buf):
        def body(r, c):
            for dr in range(4):
                for q in range(4):
                    sl = pl.ds(q * LANES, LANES)
                    buf[r * 4 + dr, sl] = buf[r * 4 + dr, sl] * SCALE
            return c

        lax.fori_loop(0, CHUNK // 4, body, 0)

    for cp in gathers(rows0, gsem0, 0):
        cp.start()

    def step(i, carry):
        @pl.when(i > 0)
        def _():
            for cp in scatters(rows1, osem1, 2 * i - 1):
                cp.wait()

        for cp in gathers(rows1, gsem1, 2 * i + 1):
            cp.start()
        for cp in gathers(rows0, gsem0, 2 * i):
            cp.wait()
        scale(rows0)
        for cp in scatters(rows0, osem0, 2 * i):
            cp.start()

        for cp in scatters(rows0, osem0, 2 * i):
            cp.wait()

        @pl.when(i < STEPS // 2 - 1)
        def _():
            for cp in gathers(rows0, gsem0, 2 * i + 2):
                cp.start()

        for cp in gathers(rows1, gsem1, 2 * i + 1):
            cp.wait()
        scale(rows1)
        for cp in scatters(rows1, osem1, 2 * i + 1):
            cp.start()
        return carry

    lax.fori_loop(0, STEPS // 2, step, 0)
    for cp in scatters(rows1, osem1, STEPS - 1):
        cp.wait()


@jax.jit
def kernel(x, lut):
    idx = x.astype(jnp.int32)
    mesh = plsc.VectorSubcoreMesh(core_axis_name="c", subcore_axis_name="s")
    run = functools.partial(
        pl.kernel,
        mesh=mesh,
        out_type=jax.ShapeDtypeStruct((BATCHES, SEQ, D_MODEL), jnp.float32),
        scratch_types=[
            pltpu.VMEM((BATCH_PER_WORKER, SEQ), jnp.int32),
            pltpu.VMEM((CHUNK, D_MODEL), jnp.float32),
            pltpu.VMEM((CHUNK, D_MODEL), jnp.float32),
            pltpu.SemaphoreType.DMA,
            pltpu.SemaphoreType.DMA,
            pltpu.SemaphoreType.DMA,
            pltpu.SemaphoreType.DMA,
        ],
        compiler_params=pltpu.CompilerParams(use_tc_tiling_on_sc=False),
    )(_emb_kernel)
    return run(idx, lut)


# speedup vs baseline: 1.1000x; 1.1000x over previous
import functools
import math

import jax
import jax.numpy as jnp
from jax import lax
from jax.experimental import pallas as pl
from jax.experimental.pallas import tpu as pltpu
from jax.experimental.pallas import tpu_sc as plsc

D_MODEL = 64
SCALE = math.sqrt(D_MODEL)
SEQ = 50
PADW = 128

NUM_CORES = 2
NUM_SUBCORES = 16
NUM_WORKERS = NUM_CORES * NUM_SUBCORES
LANES = 16

BATCHES = 16384
BATCH_PER_WORKER = BATCHES // NUM_WORKERS
NB = 4
STEPS = BATCH_PER_WORKER // NB
CHUNK = NB * SEQ


def _emb_kernel(idx_hbm, lut_hbm, out_hbm, idx0, idx1, rows0, rows1, nar0,
                nar1, gsem0, gsem1, osem0, osem1, isem0, isem1):
    wid = lax.axis_index("s") * NUM_CORES + lax.axis_index("c")
    b0 = wid * BATCH_PER_WORKER

    def stage_idx(idxbuf, sem, s):
        return pltpu.make_async_copy(
            idx_hbm.at[pl.ds(b0 + s * NB, NB)], idxbuf, sem)

    def gathers(idxbuf, buf, sem):
        return [
            pltpu.make_async_copy(lut_hbm.at[idxbuf.at[t, pl.ds(0, SEQ)]],
                                  buf.at[pl.ds(t * SEQ, SEQ)], sem)
            for t in range(NB)
        ]

    def scatters(nbuf, sem, s):
        return [
            pltpu.make_async_copy(nbuf.at[pl.ds(t * SEQ, SEQ)],
                                  out_hbm.at[b0 + s * NB + t], sem)
            for t in range(NB)
        ]

    def scale_compact(buf, nbuf):
        def body(r, c):
            for dr in range(2):
                for q in range(4):
                    sl = pl.ds(q * LANES, LANES)
                    nbuf[r * 2 + dr, sl] = buf[r * 2 + dr, sl] * SCALE
            return c

        lax.fori_loop(0, CHUNK // 2, body, 0)

    stage_idx(idx0, isem0, 0).start()
    stage_idx(idx0, isem0, 0).wait()
    stage_idx(idx1, isem1, 1).start()
    for cp in gathers(idx0, rows0, gsem0):
        cp.start()

    def step(i, carry):
        @pl.when(i > 0)
        def _():
            for cp in scatters(nar1, osem1, 2 * i - 1):
                cp.wait()

        stage_idx(idx1, isem1, 2 * i + 1).wait()
        for cp in gathers(idx1, rows1, gsem1):
            cp.start()
        for cp in gathers(idx0, rows0, gsem0):
            cp.wait()

        @pl.when(i < STEPS // 2 - 1)
        def _():
            stage_idx(idx0, isem0, 2 * i + 2).start()

        scale_compact(rows0, nar0)
        for cp in scatters(nar0, osem0, 2 * i):
            cp.start()

        for cp in scatters(nar0, osem0, 2 * i):
            cp.wait()

        @pl.when(i < STEPS // 2 - 1)
        def _():
            stage_idx(idx0, isem0, 2 * i + 2).wait()
            for cp in gathers(idx0, rows0, gsem0):
                cp.start()

        for cp in gathers(idx1, rows1, gsem1):
            cp.wait()

        @pl.when(i < STEPS // 2 - 1)
        def _():
            stage_idx(idx1, isem1, 2 * i + 3).start()

        scale_compact(rows1, nar1)
        for cp in scatters(nar1, osem1, 2 * i + 1):
            cp.start()
        return carry

    lax.fori_loop(0, STEPS // 2, step, 0)
    for cp in scatters(nar1, osem1, STEPS - 1):
        cp.wait()


@jax.jit
def kernel(x, lut):
    idx = jnp.pad(x.astype(jnp.int32), ((0, 0), (0, PADW - SEQ)))
    lut_p = jnp.pad(lut, ((0, 0), (0, PADW - D_MODEL)))
    mesh = plsc.VectorSubcoreMesh(core_axis_name="c", subcore_axis_name="s")
    run = functools.partial(
        pl.kernel,
        mesh=mesh,
        out_type=jax.ShapeDtypeStruct((BATCHES, SEQ, D_MODEL), jnp.float32),
        scratch_types=[
            pltpu.VMEM((NB, PADW), jnp.int32),
            pltpu.VMEM((NB, PADW), jnp.int32),
            pltpu.VMEM((CHUNK, PADW), jnp.float32),
            pltpu.VMEM((CHUNK, PADW), jnp.float32),
            pltpu.VMEM((CHUNK, D_MODEL), jnp.float32),
            pltpu.VMEM((CHUNK, D_MODEL), jnp.float32),
            pltpu.SemaphoreType.DMA,
            pltpu.SemaphoreType.DMA,
            pltpu.SemaphoreType.DMA,
            pltpu.SemaphoreType.DMA,
            pltpu.SemaphoreType.DMA,
            pltpu.SemaphoreType.DMA,
        ],
        compiler_params=pltpu.CompilerParams(use_tc_tiling_on_sc=True),
    )(_emb_kernel)
    return run(idx, lut_p)


# speedup vs baseline: 1.1020x; 1.0018x over previous
import functools
import math

import jax
import jax.numpy as jnp
from jax import lax
from jax.experimental import pallas as pl
from jax.experimental.pallas import tpu as pltpu
from jax.experimental.pallas import tpu_sc as plsc

D_MODEL = 64
SCALE = math.sqrt(D_MODEL)
SEQ = 50
PADW = 128
LANES = 16

NUM_CORES = 2
NUM_SUBCORES = 16
NUM_WORKERS = NUM_CORES * NUM_SUBCORES

BATCHES = 16384
BATCH_PER_WORKER = BATCHES // NUM_WORKERS
NB = 4
STEPS = BATCH_PER_WORKER // NB
CHUNK = NB * SEQ


def _emb_kernel(idx_hbm, lut_hbm, out_hbm, idx0, idx1, rows0, rows1, nar0,
                nar1, gsem0, gsem1, osem0, osem1, isem0, isem1):
    wid = lax.axis_index("s") * NUM_CORES + lax.axis_index("c")
    b0 = wid * BATCH_PER_WORKER

    def stage_idx(idxbuf, sem, s):
        return pltpu.make_async_copy(
            idx_hbm.at[pl.ds(b0 + s * NB, NB)], idxbuf, sem)

    def gathers(idxbuf, buf, sem):
        return [
            pltpu.make_async_copy(lut_hbm.at[idxbuf.at[t, pl.ds(0, SEQ)]],
                                  buf.at[pl.ds(t * SEQ, SEQ)], sem)
            for t in range(NB)
        ]

    def scatters(nbuf, sem, s):
        return [
            pltpu.make_async_copy(nbuf.at[pl.ds(t * SEQ, SEQ)],
                                  out_hbm.at[b0 + s * NB + t], sem)
            for t in range(NB)
        ]

    def scale_compact(buf, nbuf):
        def body(r, c):
            for dr in range(2):
                for q in range(D_MODEL // LANES):
                    sl = pl.ds(q * LANES, LANES)
                    nbuf[r * 2 + dr, sl] = buf[r * 2 + dr, sl] * SCALE
            return c

        lax.fori_loop(0, CHUNK // 2, body, 0)

    stage_idx(idx0, isem0, 0).start()
    stage_idx(idx0, isem0, 0).wait()
    stage_idx(idx1, isem1, 1).start()
    for cp in gathers(idx0, rows0, gsem0):
        cp.start()

    def step(i, carry):
        @pl.when(i > 0)
        def _():
            for cp in scatters(nar1, osem1, 2 * i - 1):
                cp.wait()

        stage_idx(idx1, isem1, 2 * i + 1).wait()
        for cp in gathers(idx1, rows1, gsem1):
            cp.start()
        for cp in gathers(idx0, rows0, gsem0):
            cp.wait()

        @pl.when(i < STEPS // 2 - 1)
        def _():
            stage_idx(idx0, isem0, 2 * i + 2).start()

        scale_compact(rows0, nar0)
        for cp in scatters(nar0, osem0, 2 * i):
            cp.start()

        for cp in scatters(nar0, osem0, 2 * i):
            cp.wait()

        @pl.when(i < STEPS // 2 - 1)
        def _():
            stage_idx(idx0, isem0, 2 * i + 2).wait()
            for cp in gathers(idx0, rows0, gsem0):
                cp.start()

        for cp in gathers(idx1, rows1, gsem1):
            cp.wait()

        @pl.when(i < STEPS // 2 - 1)
        def _():
            stage_idx(idx1, isem1, 2 * i + 3).start()

        scale_compact(rows1, nar1)
        for cp in scatters(nar1, osem1, 2 * i + 1):
            cp.start()
        return carry

    lax.fori_loop(0, STEPS // 2, step, 0)
    for cp in scatters(nar1, osem1, STEPS - 1):
        cp.wait()


@jax.jit
def kernel(x, lut):
    idx = jnp.pad(x.astype(jnp.int32), ((0, 0), (0, PADW - SEQ)))
    lut_p = jnp.pad(lut, ((0, 0), (0, PADW - D_MODEL)))
    mesh = plsc.VectorSubcoreMesh(core_axis_name="c", subcore_axis_name="s")
    run = functools.partial(
        pl.kernel,
        mesh=mesh,
        out_type=jax.ShapeDtypeStruct((BATCHES, SEQ, D_MODEL), jnp.float32),
        scratch_types=[
            pltpu.VMEM((NB, PADW), jnp.int32),
            pltpu.VMEM((NB, PADW), jnp.int32),
            pltpu.VMEM((CHUNK, PADW), jnp.float32),
            pltpu.VMEM((CHUNK, PADW), jnp.float32),
            pltpu.VMEM((CHUNK, D_MODEL), jnp.float32),
            pltpu.VMEM((CHUNK, D_MODEL), jnp.float32),
            pltpu.SemaphoreType.DMA,
            pltpu.SemaphoreType.DMA,
            pltpu.SemaphoreType.DMA,
            pltpu.SemaphoreType.DMA,
            pltpu.SemaphoreType.DMA,
            pltpu.SemaphoreType.DMA,
        ],
        compiler_params=pltpu.CompilerParams(use_tc_tiling_on_sc=True),
    )(_emb_kernel)
    return run(idx, lut_p)


# speedup vs baseline: 1.1021x; 1.0000x over previous
import functools
import math

import jax
import jax.numpy as jnp
from jax import lax
from jax.experimental import pallas as pl
from jax.experimental.pallas import tpu as pltpu
from jax.experimental.pallas import tpu_sc as plsc

D_MODEL = 64
SCALE = math.sqrt(D_MODEL)
SEQ = 50
PADW = 128
LANES = 16

NUM_CORES = 2
NUM_SUBCORES = 16
NUM_WORKERS = NUM_CORES * NUM_SUBCORES

BATCHES = 16384
BATCH_PER_WORKER = BATCHES // NUM_WORKERS
NB = 4
STEPS = BATCH_PER_WORKER // NB
CHUNK = NB * SEQ


def _emb_kernel(idx_hbm, lut_hbm, out_hbm, idx0, idx1, rows0, rows1, nar0,
                nar1, gsem0, gsem1, osem0, osem1, isem0, isem1):
    wid = lax.axis_index("s") * NUM_CORES + lax.axis_index("c")
    b0 = wid * BATCH_PER_WORKER

    def stage_idx(idxbuf, sem, s):
        return pltpu.make_async_copy(
            idx_hbm.at[pl.ds(b0 + s * NB, NB)], idxbuf, sem)

    def gathers(idxbuf, buf, sem):
        return [
            pltpu.make_async_copy(lut_hbm.at[idxbuf.at[t, pl.ds(0, SEQ)]],
                                  buf.at[pl.ds(t * SEQ, SEQ)], sem)
            for t in range(NB)
        ]

    def scatters(nbuf, sem, s):
        return [
            pltpu.make_async_copy(nbuf.at[pl.ds(t * SEQ, SEQ)],
                                  out_hbm.at[b0 + s * NB + t], sem)
            for t in range(NB)
        ]

    def scale_block(buf, nbuf, t):
        def body(r, c):
            for dr in range(2):
                for q in range(D_MODEL // LANES):
                    sl = pl.ds(q * LANES, LANES)
                    nbuf[t * SEQ + r * 2 + dr, sl] = (
                        buf[t * SEQ + r * 2 + dr, sl] * SCALE)
            return c

        lax.fori_loop(0, SEQ // 2, body, 0)

    def scatter_one(nbuf, sem, s, t):
        return pltpu.make_async_copy(nbuf.at[pl.ds(t * SEQ, SEQ)],
                                     out_hbm.at[b0 + s * NB + t], sem)

    def scale_scatter(buf, nbuf, sem, s):
        for t in range(NB):
            scale_block(buf, nbuf, t)
            scatter_one(nbuf, sem, s, t).start()

    stage_idx(idx0, isem0, 0).start()
    stage_idx(idx0, isem0, 0).wait()
    stage_idx(idx1, isem1, 1).start()
    for cp in gathers(idx0, rows0, gsem0):
        cp.start()

    def step(i, carry):
        @pl.when(i > 0)
        def _():
            for cp in scatters(nar1, osem1, 2 * i - 1):
                cp.wait()

        stage_idx(idx1, isem1, 2 * i + 1).wait()
        for cp in gathers(idx1, rows1, gsem1):
            cp.start()
        for cp in gathers(idx0, rows0, gsem0):
            cp.wait()

        @pl.when(i < STEPS // 2 - 1)
        def _():
            stage_idx(idx0, isem0, 2 * i + 2).start()

        scale_scatter(rows0, nar0, osem0, 2 * i)

        for cp in scatters(nar0, osem0, 2 * i):
            cp.wait()

        @pl.when(i < STEPS // 2 - 1)
        def _():
            stage_idx(idx0, isem0, 2 * i + 2).wait()
            for cp in gathers(idx0, rows0, gsem0):
                cp.start()

        for cp in gathers(idx1, rows1, gsem1):
            cp.wait()

        @pl.when(i < STEPS // 2 - 1)
        def _():
            stage_idx(idx1, isem1, 2 * i + 3).start()

        scale_scatter(rows1, nar1, osem1, 2 * i + 1)
        return carry

    lax.fori_loop(0, STEPS // 2, step, 0)
    for cp in scatters(nar1, osem1, STEPS - 1):
        cp.wait()


@jax.jit
def kernel(x, lut):
    idx = jnp.pad(x.astype(jnp.int32), ((0, 0), (0, PADW - SEQ)))
    lut_p = jnp.pad(lut, ((0, 0), (0, PADW - D_MODEL)))
    mesh = plsc.VectorSubcoreMesh(core_axis_name="c", subcore_axis_name="s")
    run = functools.partial(
        pl.kernel,
        mesh=mesh,
        out_type=jax.ShapeDtypeStruct((BATCHES, SEQ, D_MODEL), jnp.float32),
        scratch_types=[
            pltpu.VMEM((NB, PADW), jnp.int32),
            pltpu.VMEM((NB, PADW), jnp.int32),
            pltpu.VMEM((CHUNK, PADW), jnp.float32),
            pltpu.VMEM((CHUNK, PADW), jnp.float32),
            pltpu.VMEM((CHUNK, D_MODEL), jnp.float32),
            pltpu.VMEM((CHUNK, D_MODEL), jnp.float32),
            pltpu.SemaphoreType.DMA,
            pltpu.SemaphoreType.DMA,
            pltpu.SemaphoreType.DMA,
            pltpu.SemaphoreType.DMA,
            pltpu.SemaphoreType.DMA,
            pltpu.SemaphoreType.DMA,
        ],
        compiler_params=pltpu.CompilerParams(use_tc_tiling_on_sc=True),
    )(_emb_kernel)
    return run(idx, lut_p)
